# baseline (speedup 1.0000x reference)
import functools

import jax
import jax.numpy as jnp
from jax import lax
from jax.experimental import pallas as pl
from jax.experimental.pallas import tpu as pltpu
from jax.experimental.pallas import tpu_sc as plsc

N = 10000
E = 160000
HH = 64
D1 = 128
D2 = 128
NS = 16
CHUNK = 80
EPT = E // NS
NCH = EPT // CHUNK
STRIPE = 624
TAIL = N - NS * STRIPE


def _bdot(a, b):
    return jnp.dot(a, b.astype(jnp.bfloat16), preferred_element_type=jnp.float32)


def _tc1_body(x_ref, wpl_ref, wnl_ref, wpr_ref, wnr_ref, bp_ref, bn_ref,
              up_ref, un_ref, r1_ref):
    x = x_ref[...]
    xb = x.astype(jnp.bfloat16)
    n = x.shape[0]
    pad = jnp.concatenate(
        [jnp.ones((n, 1), jnp.float32), jnp.zeros((n, D1 - HH - 1), jnp.float32)],
        axis=1)
    up_ref[...] = jnp.concatenate([_bdot(xb, wpl_ref[...]), pad], axis=1)
    un_ref[...] = jnp.concatenate([_bdot(xb, wnl_ref[...]), pad], axis=1)
    wr = jnp.concatenate([wpr_ref[...], wnr_ref[...]], axis=1)
    br = jnp.concatenate([bp_ref[...], bn_ref[...]], axis=1)
    r1_ref[...] = _bdot(xb, wr) + br


def _tc2_body(sump_ref, sumn_ref, r1_ref, wpl_ref, wpr_ref, bp_ref,
              wnl_ref, wnr_ref, bn_ref, ac_ref, bd_ref, r2_ref):
    cp = jnp.maximum(sump_ref[:, 64:65], 1.0)
    cn = jnp.maximum(sumn_ref[:, 64:65], 1.0)
    zp = jnp.tanh(sump_ref[:, :64] / cp + r1_ref[:, :64])
    zn = jnp.tanh(sumn_ref[:, :64] / cn + r1_ref[:, 64:])
    zpb = zp.astype(jnp.bfloat16)
    znb = zn.astype(jnp.bfloat16)
    wpl = wpl_ref[...]
    wnl = wnl_ref[...]
    ac_ref[...] = jnp.concatenate(
        [_bdot(zpb, wpl[:HH]), _bdot(znb, wnl[:HH])], axis=1)
    bd_ref[...] = jnp.concatenate(
        [_bdot(znb, wpl[HH:]), _bdot(zpb, wnl[HH:])], axis=1)
    r2_ref[...] = jnp.concatenate(
        [_bdot(zpb, wpr_ref[...]) + bp_ref[...],
         _bdot(znb, wnr_ref[...]) + bn_ref[...]], axis=1)


def _tc3_body(sac_ref, sbd_ref, r2_ref, sump_ref, sumn_ref, out_ref):
    cp = jnp.maximum(sump_ref[:, 64:65], 1.0)
    cn = jnp.maximum(sumn_ref[:, 64:65], 1.0)
    out_ref[...] = jnp.tanh(sac_ref[...] / cp + sbd_ref[...] / cn + r2_ref[...])


_f32 = lambda *s: jax.ShapeDtypeStruct(s, jnp.float32)

_tc1 = pl.pallas_call(_tc1_body, out_shape=(_f32(N, D1), _f32(N, D1), _f32(N, D2)))
_tc2 = pl.pallas_call(_tc2_body, out_shape=(_f32(N, D2), _f32(N, D2), _f32(N, D2)))
_tc3 = pl.pallas_call(_tc3_body, out_shape=_f32(N, D2))


@functools.lru_cache(maxsize=None)
def _make_seg(D):
    mesh = plsc.VectorSubcoreMesh(core_axis_name="c", subcore_axis_name="s")

    @functools.partial(
        pl.kernel, mesh=mesh,
        out_type=[_f32(N, D), _f32(N, D)],
        scratch_types=[
            pltpu.VMEM((EPT,), jnp.int32),
            pltpu.VMEM((1, CHUNK), jnp.int32),
            pltpu.VMEM((1, CHUNK), jnp.int32),
            pltpu.VMEM((1, CHUNK), jnp.int32),
            pltpu.VMEM((CHUNK, D), jnp.float32),
            pltpu.VMEM((CHUNK, D), jnp.float32),
            pltpu.VMEM((CHUNK, D), jnp.float32),
            pltpu.VMEM_SHARED((N, D), jnp.float32),
            pltpu.SemaphoreType.DMA,
            pltpu.SemaphoreType.DMA,
            pltpu.SemaphoreType.DMA,
            pltpu.SemaphoreType.DMA,
            pltpu.SemaphoreType.DMA,
            pltpu.SemaphoreType.DMA,
            pltpu.SemaphoreType.DMA,
            pltpu.SemaphoreType.DMA,
            pltpu.SemaphoreType.DMA,
        ],
    )
    def seg(featp, featn, srcp, dstp, srcn, dstn, zeros_hbm,
            outp, outn, sidx, didx0, didx1, didx2, rows0, rows1, rows2, acc,
            sem0, sem1, sem2, semi0, semi1, semi2, sems0, sems1, sems2):
        c = lax.axis_index("c")
        s = lax.axis_index("s")
        stripe = pl.ds(s * STRIPE, STRIPE)
        tail = pl.ds(NS * STRIPE, TAIL)
        pltpu.sync_copy(zeros_hbm.at[stripe], acc.at[stripe])

        @pl.when(s == 0)
        def _():
            pltpu.sync_copy(zeros_hbm.at[tail], acc.at[tail])

        plsc.subcore_barrier()

        def run(src, dst, feat):
            pltpu.sync_copy(src.at[pl.ds(s * EPT, EPT)], sidx)
            didx = (didx0, didx1, didx2)
            rows = (rows0, rows1, rows2)
            semg = (sem0, sem1, sem2)
            semi = (semi0, semi1, semi2)
            sems = (sems0, sems1, sems2)

            def issue(i, b):
                pltpu.async_copy(dst.at[s * NCH + i], didx[b], semi[b])
                pltpu.async_copy(
                    feat.at[sidx.at[pl.ds(i * CHUNK, CHUNK)]], rows[b], semg[b])

            def scat_wait(i, b):
                pltpu.make_async_copy(
                    rows[b], acc.at[didx[b].at[0]], sems[b]).wait()

            def core(i, b, guard_first, guard_issue):
                pltpu.make_async_copy(
                    feat.at[sidx.at[pl.ds(i * CHUNK, CHUNK)]], rows[b],
                    semg[b]).wait()
                pltpu.make_async_copy(dst.at[s * NCH + i], didx[b], semi[b]).wait()
                pltpu.async_copy(rows[b], acc.at[didx[b].at[0]], sems[b], add=True)
                if guard_first:
                    @pl.when(i >= 1)
                    def _():
                        scat_wait(i - 1, (b + 2) % 3)
                else:
                    scat_wait(i - 1, (b + 2) % 3)
                if guard_issue:
                    @pl.when(i + 2 < NCH)
                    def _():
                        issue(i + 2, (b + 2) % 3)
                else:
                    issue(i + 2, (b + 2) % 3)

            issue(0, 0)
            issue(1, 1)

            def body(j, carry):
                i = 3 * j
                core(i, 0, guard_first=True, guard_issue=True)
                core(i + 1, 1, guard_first=False, guard_issue=True)
                core(i + 2, 2, guard_first=False, guard_issue=True)
                return carry

            lax.fori_loop(0, NCH // 3, body, 0)
            for i in range(NCH - NCH % 3, NCH):
                b = i % 3
                pltpu.make_async_copy(
                    feat.at[sidx.at[pl.ds(i * CHUNK, CHUNK)]], rows[b],
                    semg[b]).wait()
                pltpu.make_async_copy(dst.at[s * NCH + i], didx[b], semi[b]).wait()
                pltpu.async_copy(rows[b], acc.at[didx[b].at[0]], sems[b], add=True)
                scat_wait(i - 1, (i - 1) % 3)
            scat_wait(NCH - 1, (NCH - 1) % 3)

        @pl.when(c == 0)
        def _():
            run(srcp, dstp, featp)

        @pl.when(c == 1)
        def _():
            run(srcn, dstn, featn)

        plsc.subcore_barrier()

        @pl.when(c == 0)
        def _():
            pltpu.sync_copy(acc.at[stripe], outp.at[stripe])

            @pl.when(s == 0)
            def _():
                pltpu.sync_copy(acc.at[tail], outp.at[tail])

        @pl.when(c == 1)
        def _():
            pltpu.sync_copy(acc.at[stripe], outn.at[stripe])

            @pl.when(s == 0)
            def _():
                pltpu.sync_copy(acc.at[tail], outn.at[tail])

    return seg


def kernel(x, pos_edge_index, neg_edge_index,
           c1_wpl, c1_wpr, c1_bp, c1_wnl, c1_wnr, c1_bn,
           c2_wpl, c2_wpr, c2_bp, c2_wnl, c2_wnr, c2_bn):
    srcp = pos_edge_index[0].astype(jnp.int32)
    dstp = pos_edge_index[1].astype(jnp.int32).reshape(NS * NCH, 1, CHUNK)
    srcn = neg_edge_index[0].astype(jnp.int32)
    dstn = neg_edge_index[1].astype(jnp.int32).reshape(NS * NCH, 1, CHUNK)

    bp = c1_bp.reshape(1, HH)
    bn = c1_bn.reshape(1, HH)
    bp2 = c2_bp.reshape(1, HH)
    bn2 = c2_bn.reshape(1, HH)
    zeros = jnp.zeros((N, D2), jnp.float32)

    up, un, r1 = _tc1(x, c1_wpl, c1_wnl, c1_wpr, c1_wnr, bp, bn)
    sump, sumn = _make_seg(D1)(up, un, srcp, dstp, srcn, dstn, zeros)
    ac, bd, r2 = _tc2(sump, sumn, r1, c2_wpl, c2_wpr, bp2, c2_wnl, c2_wnr, bn2)
    sac, sbd = _make_seg(D2)(ac, bd, srcp, dstp, srcn, dstn, zeros)
    return _tc3(sac, sbd, r2, sump, sumn)


# speedup vs baseline: 1.0037x; 1.0037x over previous
import functools

import jax
import jax.numpy as jnp
from jax import lax
from jax.experimental import pallas as pl
from jax.experimental.pallas import tpu as pltpu
from jax.experimental.pallas import tpu_sc as plsc

N = 10000
E = 160000
HH = 64
D1 = 128
D2 = 128
NS = 16
CHUNK = 80
EPT = E // NS
NCH = EPT // CHUNK
STRIPE = 624
TAIL = N - NS * STRIPE


def _bdot(a, b):
    return jnp.dot(a, b.astype(jnp.bfloat16), preferred_element_type=jnp.float32)


def _tc1_body(x_ref, wp_ref, wn_ref, wr_ref, bp80_ref, bn80_ref, br_ref,
              up_ref, un_ref, r1_ref):
    xb = x_ref[...].astype(jnp.bfloat16)
    up_ref[...] = _bdot(xb, wp_ref[...]) + bp80_ref[...]
    un_ref[...] = _bdot(xb, wn_ref[...]) + bn80_ref[...]
    r1_ref[...] = _bdot(xb, wr_ref[...]) + br_ref[...]


def _tc2_body(sump_ref, sumn_ref, r1_ref, wac_ref, wbd_ref, wr2_ref, br2_ref,
              ac_ref, bd_ref, r2_ref):
    cp = jnp.maximum(sump_ref[:, 64:65], 1.0)
    cn = jnp.maximum(sumn_ref[:, 64:65], 1.0)
    zp = jnp.tanh(sump_ref[:, :64] / cp + r1_ref[:, :64])
    zn = jnp.tanh(sumn_ref[:, :64] / cn + r1_ref[:, 64:])
    z = jnp.concatenate([zp, zn], axis=1).astype(jnp.bfloat16)
    ac_ref[...] = _bdot(z, wac_ref[...])
    bd_ref[...] = _bdot(z, wbd_ref[...])
    r2_ref[...] = _bdot(z, wr2_ref[...]) + br2_ref[...]


def _tc3_body(sac_ref, sbd_ref, r2_ref, sump_ref, sumn_ref, out_ref):
    cp = jnp.maximum(sump_ref[:, 64:65], 1.0)
    cn = jnp.maximum(sumn_ref[:, 64:65], 1.0)
    out_ref[...] = jnp.tanh(sac_ref[...] / cp + sbd_ref[...] / cn + r2_ref[...])


_f32 = lambda *s: jax.ShapeDtypeStruct(s, jnp.float32)

_tc1 = pl.pallas_call(_tc1_body, out_shape=(_f32(N, D1), _f32(N, D1), _f32(N, D2)))
_tc2 = pl.pallas_call(_tc2_body, out_shape=(_f32(N, D2), _f32(N, D2), _f32(N, D2)))
_tc3 = pl.pallas_call(_tc3_body, out_shape=_f32(N, D2))


@functools.lru_cache(maxsize=None)
def _make_seg(D):
    mesh = plsc.VectorSubcoreMesh(core_axis_name="c", subcore_axis_name="s")

    @functools.partial(
        pl.kernel, mesh=mesh,
        out_type=[_f32(N, D), _f32(N, D)],
        scratch_types=[
            pltpu.VMEM((EPT,), jnp.int32),
            pltpu.VMEM((1, CHUNK), jnp.int32),
            pltpu.VMEM((1, CHUNK), jnp.int32),
            pltpu.VMEM((1, CHUNK), jnp.int32),
            pltpu.VMEM((CHUNK, D), jnp.float32),
            pltpu.VMEM((CHUNK, D), jnp.float32),
            pltpu.VMEM((CHUNK, D), jnp.float32),
            pltpu.VMEM_SHARED((N, D), jnp.float32),
            pltpu.SemaphoreType.DMA,
            pltpu.SemaphoreType.DMA,
            pltpu.SemaphoreType.DMA,
            pltpu.SemaphoreType.DMA,
            pltpu.SemaphoreType.DMA,
            pltpu.SemaphoreType.DMA,
            pltpu.SemaphoreType.DMA,
            pltpu.SemaphoreType.DMA,
            pltpu.SemaphoreType.DMA,
        ],
    )
    def seg(featp, featn, srcp, dstp, srcn, dstn, zeros_hbm,
            outp, outn, sidx, didx0, didx1, didx2, rows0, rows1, rows2, acc,
            sem0, sem1, sem2, semi0, semi1, semi2, sems0, sems1, sems2):
        c = lax.axis_index("c")
        s = lax.axis_index("s")
        stripe = pl.ds(s * STRIPE, STRIPE)
        tail = pl.ds(NS * STRIPE, TAIL)
        pltpu.sync_copy(zeros_hbm.at[stripe], acc.at[stripe])

        @pl.when(s == 0)
        def _():
            pltpu.sync_copy(zeros_hbm.at[tail], acc.at[tail])

        plsc.subcore_barrier()

        def run(src, dst, feat):
            pltpu.sync_copy(src.at[pl.ds(s * EPT, EPT)], sidx)
            didx = (didx0, didx1, didx2)
            rows = (rows0, rows1, rows2)
            semg = (sem0, sem1, sem2)
            semi = (semi0, semi1, semi2)
            sems = (sems0, sems1, sems2)

            def issue(i, b):
                pltpu.async_copy(dst.at[s * NCH + i], didx[b], semi[b])
                pltpu.async_copy(
                    feat.at[sidx.at[pl.ds(i * CHUNK, CHUNK)]], rows[b], semg[b])

            def scat_wait(i, b):
                pltpu.make_async_copy(
                    rows[b], acc.at[didx[b].at[0]], sems[b]).wait()

            def core(i, b, guard_first, guard_issue):
                pltpu.make_async_copy(
                    feat.at[sidx.at[pl.ds(i * CHUNK, CHUNK)]], rows[b],
                    semg[b]).wait()
                pltpu.make_async_copy(dst.at[s * NCH + i], didx[b], semi[b]).wait()
                pltpu.async_copy(rows[b], acc.at[didx[b].at[0]], sems[b], add=True)
                if guard_first:
                    @pl.when(i >= 1)
                    def _():
                        scat_wait(i - 1, (b + 2) % 3)
                else:
                    scat_wait(i - 1, (b + 2) % 3)
                if guard_issue:
                    @pl.when(i + 2 < NCH)
                    def _():
                        issue(i + 2, (b + 2) % 3)
                else:
                    issue(i + 2, (b + 2) % 3)

            issue(0, 0)
            issue(1, 1)

            def body(j, carry):
                i = 3 * j
                core(i, 0, guard_first=True, guard_issue=True)
                core(i + 1, 1, guard_first=False, guard_issue=True)
                core(i + 2, 2, guard_first=False, guard_issue=True)
                return carry

            lax.fori_loop(0, NCH // 3, body, 0)
            for i in range(NCH - NCH % 3, NCH):
                b = i % 3
                pltpu.make_async_copy(
                    feat.at[sidx.at[pl.ds(i * CHUNK, CHUNK)]], rows[b],
                    semg[b]).wait()
                pltpu.make_async_copy(dst.at[s * NCH + i], didx[b], semi[b]).wait()
                pltpu.async_copy(rows[b], acc.at[didx[b].at[0]], sems[b], add=True)
                scat_wait(i - 1, (i - 1) % 3)
            scat_wait(NCH - 1, (NCH - 1) % 3)

        @pl.when(c == 0)
        def _():
            run(srcp, dstp, featp)

        @pl.when(c == 1)
        def _():
            run(srcn, dstn, featn)

        plsc.subcore_barrier()

        @pl.when(c == 0)
        def _():
            pltpu.sync_copy(acc.at[stripe], outp.at[stripe])

            @pl.when(s == 0)
            def _():
                pltpu.sync_copy(acc.at[tail], outp.at[tail])

        @pl.when(c == 1)
        def _():
            pltpu.sync_copy(acc.at[stripe], outn.at[stripe])

            @pl.when(s == 0)
            def _():
                pltpu.sync_copy(acc.at[tail], outn.at[tail])

    return seg


def kernel(x, pos_edge_index, neg_edge_index,
           c1_wpl, c1_wpr, c1_bp, c1_wnl, c1_wnr, c1_bn,
           c2_wpl, c2_wpr, c2_bp, c2_wnl, c2_wnr, c2_bn):
    srcp = pos_edge_index[0].astype(jnp.int32)
    dstp = pos_edge_index[1].astype(jnp.int32).reshape(NS * NCH, 1, CHUNK)
    srcn = neg_edge_index[0].astype(jnp.int32)
    dstn = neg_edge_index[1].astype(jnp.int32).reshape(NS * NCH, 1, CHUNK)

    pad = jnp.zeros((D2, D1 - HH), jnp.float32)
    wp80 = jnp.concatenate([c1_wpl, pad], axis=1)
    wn80 = jnp.concatenate([c1_wnl, pad], axis=1)
    onescol = jnp.concatenate([jnp.zeros((1, HH), jnp.float32),
                               jnp.ones((1, 1), jnp.float32),
                               jnp.zeros((1, D1 - HH - 1), jnp.float32)], axis=1)
    w1r = jnp.concatenate([c1_wpr, c1_wnr], axis=1)
    b1r = jnp.concatenate([c1_bp, c1_bn]).reshape(1, D2)

    z64 = jnp.zeros((HH, HH), jnp.float32)
    wac = jnp.block([[c2_wpl[:HH], z64], [z64, c2_wnl[:HH]]])
    wbd = jnp.block([[z64, c2_wnl[HH:]], [c2_wpl[HH:], z64]])
    wr2 = jnp.block([[c2_wpr, z64], [z64, c2_wnr]])
    br2 = jnp.concatenate([c2_bp, c2_bn]).reshape(1, D2)

    zeros = jnp.zeros((N, D2), jnp.float32)

    up, un, r1 = _tc1(x, wp80, wn80, w1r, onescol, onescol, b1r)
    sump, sumn = _make_seg(D1)(up, un, srcp, dstp, srcn, dstn, zeros)
    ac, bd, r2 = _tc2(sump, sumn, r1, wac, wbd, wr2, br2)
    sac, sbd = _make_seg(D2)(ac, bd, srcp, dstp, srcn, dstn, zeros)
    return _tc3(sac, sbd, r2, sump, sumn)


# speedup vs baseline: 1.0177x; 1.0139x over previous
import functools

import jax
import jax.numpy as jnp
from jax import lax
from jax.experimental import pallas as pl
from jax.experimental.pallas import tpu as pltpu
from jax.experimental.pallas import tpu_sc as plsc

N = 10000
E = 160000
HH = 64
D1 = 128
D2 = 128
NS = 16
CHUNK = 80
EPT = E // NS
NCH = EPT // CHUNK
STRIPE = 624
TAIL = N - NS * STRIPE


def _tc1_body(x_ref, wp_ref, wn_ref, wr_ref, bp80_ref, bn80_ref, br_ref,
              up_ref, un_ref, r1_ref):
    x = x_ref[...]
    up_ref[...] = jnp.dot(x, wp_ref[...], preferred_element_type=jnp.float32) + bp80_ref[...]
    un_ref[...] = jnp.dot(x, wn_ref[...], preferred_element_type=jnp.float32) + bn80_ref[...]
    r1_ref[...] = jnp.dot(x, wr_ref[...], preferred_element_type=jnp.float32) + br_ref[...]


def _tc2_body(sump_ref, sumn_ref, r1_ref, wac_ref, wbd_ref, wr2_ref, br2_ref,
              ac_ref, bd_ref, r2_ref, cnt_ref):
    cp = jnp.maximum(sump_ref[:, 64:65], 1.0)
    cn = jnp.maximum(sumn_ref[:, 64:65], 1.0)
    zp = jnp.tanh(sump_ref[:, :64] / cp + r1_ref[:, :64])
    zn = jnp.tanh(sumn_ref[:, :64] / cn + r1_ref[:, 64:])
    z = jnp.concatenate([zp, zn], axis=1)
    ac_ref[...] = jnp.dot(z, wac_ref[...], preferred_element_type=jnp.float32)
    bd_ref[...] = jnp.dot(z, wbd_ref[...], preferred_element_type=jnp.float32)
    r2_ref[...] = jnp.dot(z, wr2_ref[...], preferred_element_type=jnp.float32) + br2_ref[...]
    cnt_ref[...] = jnp.concatenate(
        [cp, cn, jnp.zeros((cp.shape[0], 6), jnp.float32)], axis=1)


def _tc3_body(sac_ref, sbd_ref, r2_ref, cnt_ref, out_ref):
    cp = cnt_ref[:, 0:1]
    cn = cnt_ref[:, 1:2]
    out_ref[...] = jnp.tanh(sac_ref[...] / cp + sbd_ref[...] / cn + r2_ref[...])


_f32 = lambda *s: jax.ShapeDtypeStruct(s, jnp.float32)

_tc1 = pl.pallas_call(_tc1_body, out_shape=(_f32(N, D1), _f32(N, D1), _f32(N, D2)))
_tc2 = pl.pallas_call(_tc2_body, out_shape=(_f32(N, D2), _f32(N, D2), _f32(N, D2), _f32(N, 8)))
_tc3 = pl.pallas_call(_tc3_body, out_shape=_f32(N, D2))


@functools.lru_cache(maxsize=None)
def _make_seg(D):
    mesh = plsc.VectorSubcoreMesh(core_axis_name="c", subcore_axis_name="s")

    @functools.partial(
        pl.kernel, mesh=mesh,
        out_type=[_f32(N, D), _f32(N, D)],
        scratch_types=[
            pltpu.VMEM((EPT,), jnp.int32),
            pltpu.VMEM((1, CHUNK), jnp.int32),
            pltpu.VMEM((1, CHUNK), jnp.int32),
            pltpu.VMEM((1, CHUNK), jnp.int32),
            pltpu.VMEM((CHUNK, D), jnp.float32),
            pltpu.VMEM((CHUNK, D), jnp.float32),
            pltpu.VMEM((CHUNK, D), jnp.float32),
            pltpu.VMEM_SHARED((N, D), jnp.float32),
            pltpu.SemaphoreType.DMA,
            pltpu.SemaphoreType.DMA,
            pltpu.SemaphoreType.DMA,
            pltpu.SemaphoreType.DMA,
            pltpu.SemaphoreType.DMA,
            pltpu.SemaphoreType.DMA,
            pltpu.SemaphoreType.DMA,
            pltpu.SemaphoreType.DMA,
            pltpu.SemaphoreType.DMA,
        ],
    )
    def seg(featp, featn, srcp, dstp, srcn, dstn, zeros_hbm,
            outp, outn, sidx, didx0, didx1, didx2, rows0, rows1, rows2, acc,
            sem0, sem1, sem2, semi0, semi1, semi2, sems0, sems1, sems2):
        c = lax.axis_index("c")
        s = lax.axis_index("s")
        stripe = pl.ds(s * STRIPE, STRIPE)
        tail = pl.ds(NS * STRIPE, TAIL)

        def run(src, dst, feat):
            pltpu.sync_copy(src.at[pl.ds(s * EPT, EPT)], sidx)
            didx = (didx0, didx1, didx2)
            rows = (rows0, rows1, rows2)
            semg = (sem0, sem1, sem2)
            semi = (semi0, semi1, semi2)
            sems = (sems0, sems1, sems2)

            def issue(i, b):
                pltpu.async_copy(dst.at[s * NCH + i], didx[b], semi[b])
                pltpu.async_copy(
                    feat.at[sidx.at[pl.ds(i * CHUNK, CHUNK)]], rows[b], semg[b])

            def scat_wait(i, b):
                pltpu.make_async_copy(
                    rows[b], acc.at[didx[b].at[0]], sems[b]).wait()

            def core(i, b, guard_first, guard_issue):
                pltpu.make_async_copy(
                    feat.at[sidx.at[pl.ds(i * CHUNK, CHUNK)]], rows[b],
                    semg[b]).wait()
                pltpu.make_async_copy(dst.at[s * NCH + i], didx[b], semi[b]).wait()
                pltpu.async_copy(rows[b], acc.at[didx[b].at[0]], sems[b], add=True)
                if guard_first:
                    @pl.when(i >= 1)
                    def _():
                        scat_wait(i - 1, (b + 2) % 3)
                else:
                    scat_wait(i - 1, (b + 2) % 3)
                if guard_issue:
                    @pl.when(i + 2 < NCH)
                    def _():
                        issue(i + 2, (b + 2) % 3)
                else:
                    issue(i + 2, (b + 2) % 3)

            issue(0, 0)
            issue(1, 1)
            pltpu.sync_copy(zeros_hbm.at[stripe], acc.at[stripe])

            @pl.when(s == 0)
            def _():
                pltpu.sync_copy(zeros_hbm.at[tail], acc.at[tail])

            plsc.subcore_barrier()

            def body(j, carry):
                i = 3 * j
                core(i, 0, guard_first=True, guard_issue=True)
                core(i + 1, 1, guard_first=False, guard_issue=True)
                core(i + 2, 2, guard_first=False, guard_issue=True)
                return carry

            lax.fori_loop(0, NCH // 3, body, 0)
            for i in range(NCH - NCH % 3, NCH):
                b = i % 3
                pltpu.make_async_copy(
                    feat.at[sidx.at[pl.ds(i * CHUNK, CHUNK)]], rows[b],
                    semg[b]).wait()
                pltpu.make_async_copy(dst.at[s * NCH + i], didx[b], semi[b]).wait()
                pltpu.async_copy(rows[b], acc.at[didx[b].at[0]], sems[b], add=True)
                scat_wait(i - 1, (i - 1) % 3)
            scat_wait(NCH - 1, (NCH - 1) % 3)

        @pl.when(c == 0)
        def _():
            run(srcp, dstp, featp)

        @pl.when(c == 1)
        def _():
            run(srcn, dstn, featn)

        plsc.subcore_barrier()

        @pl.when(c == 0)
        def _():
            pltpu.sync_copy(acc.at[stripe], outp.at[stripe])

            @pl.when(s == 0)
            def _():
                pltpu.sync_copy(acc.at[tail], outp.at[tail])

        @pl.when(c == 1)
        def _():
            pltpu.sync_copy(acc.at[stripe], outn.at[stripe])

            @pl.when(s == 0)
            def _():
                pltpu.sync_copy(acc.at[tail], outn.at[tail])

    return seg


def kernel(x, pos_edge_index, neg_edge_index,
           c1_wpl, c1_wpr, c1_bp, c1_wnl, c1_wnr, c1_bn,
           c2_wpl, c2_wpr, c2_bp, c2_wnl, c2_wnr, c2_bn):
    srcp = pos_edge_index[0].astype(jnp.int32)
    dstp = pos_edge_index[1].astype(jnp.int32).reshape(NS * NCH, 1, CHUNK)
    srcn = neg_edge_index[0].astype(jnp.int32)
    dstn = neg_edge_index[1].astype(jnp.int32).reshape(NS * NCH, 1, CHUNK)

    pad = jnp.zeros((D2, D1 - HH), jnp.float32)
    wp80 = jnp.concatenate([c1_wpl, pad], axis=1)
    wn80 = jnp.concatenate([c1_wnl, pad], axis=1)
    onescol = jnp.concatenate([jnp.zeros((1, HH), jnp.float32),
                               jnp.ones((1, 1), jnp.float32),
                               jnp.zeros((1, D1 - HH - 1), jnp.float32)], axis=1)
    w1r = jnp.concatenate([c1_wpr, c1_wnr], axis=1)
    b1r = jnp.concatenate([c1_bp, c1_bn]).reshape(1, D2)

    z64 = jnp.zeros((HH, HH), jnp.float32)
    wac = jnp.block([[c2_wpl[:HH], z64], [z64, c2_wnl[:HH]]])
    wbd = jnp.block([[z64, c2_wnl[HH:]], [c2_wpl[HH:], z64]])
    wr2 = jnp.block([[c2_wpr, z64], [z64, c2_wnr]])
    br2 = jnp.concatenate([c2_bp, c2_bn]).reshape(1, D2)

    zeros = jnp.zeros((N, D2), jnp.float32)

    up, un, r1 = _tc1(x, wp80, wn80, w1r, onescol, onescol, b1r)
    sump, sumn = _make_seg(D1)(up, un, srcp, dstp, srcn, dstn, zeros)
    ac, bd, r2, cnt = _tc2(sump, sumn, r1, wac, wbd, wr2, br2)
    sac, sbd = _make_seg(D2)(ac, bd, srcp, dstp, srcn, dstn, zeros)
    return _tc3(sac, sbd, r2, cnt)
